# baseline (speedup 1.0000x reference)
import functools

import jax
import jax.numpy as jnp
from jax import lax
from jax.experimental import pallas as pl
from jax.experimental.pallas import tpu as pltpu
from jax.experimental.pallas import tpu_sc as plsc

T, V, C, H, DS, E, OUT = 4, 10000, 128, 64, 16, 160000, 128
HDS = H * DS

NS = 16
K = 512
EP = 10240
EPAD = EP * NS
NCHUNK = EP // K

_MM = dict(preferred_element_type=jnp.float32,
           precision=jax.lax.Precision.DEFAULT)


@functools.cache
def _make_sc_agg(with_deg):
  mesh = plsc.VectorSubcoreMesh(core_axis_name="c", subcore_axis_name="s")
  out_type = [jax.ShapeDtypeStruct((T, V, H), jnp.float32)]
  if with_deg:
    out_type.append(jax.ShapeDtypeStruct((T, V, 8), jnp.float32))
  scratch = [
      pltpu.VMEM((K,), jnp.int32),
      pltpu.VMEM((K,), jnp.int32),
      pltpu.VMEM((K,), jnp.int32),
      pltpu.VMEM((K, H), jnp.float32),
      pltpu.VMEM((K, 8), jnp.float32),
      pltpu.SemaphoreType.DMA,
      pltpu.VMEM_SHARED((V + 8, H), jnp.float32),
      pltpu.VMEM_SHARED((V + 8, 8), jnp.float32),
  ]

  def body(feat_h, pk_h, zrows_h, zdeg_h, ones_h, *rest):
    if with_deg:
      agg_h, deg_h = rest[0], rest[1]
      rest = rest[2:]
    else:
      agg_h = rest[0]
      rest = rest[1:]
    idx_pk, idx_s, idx_d, rows, ones_v, sem, agg_sh, deg_sh = rest
    c = lax.axis_index("c")
    s = lax.axis_index("s")

    if with_deg:
      pltpu.sync_copy(ones_h, ones_v)

    for tl in range(2):
      t = 2 * c + tl

      @pl.when(s < 10)
      def _():
        for k2 in range(8):
          pltpu.sync_copy(zrows_h,
                          agg_sh.at[pl.ds(s * 1000 + k2 * 125, 125)])
          if with_deg:
            pltpu.sync_copy(zdeg_h,
                            deg_sh.at[pl.ds(s * 1000 + k2 * 125, 125)])
      @pl.when(s == 10)
      def _():
        pltpu.sync_copy(zrows_h.at[pl.ds(0, 8)], agg_sh.at[pl.ds(V, 8)])
        if with_deg:
          pltpu.sync_copy(zdeg_h.at[pl.ds(0, 8)], deg_sh.at[pl.ds(V, 8)])
      plsc.subcore_barrier()

      def chunk(i, _):
        base = t * EPAD + s * EP + i * K
        pltpu.sync_copy(pk_h.at[pl.ds(base, K)], idx_pk)

        def unpack(j, _2):
          v = idx_pk[pl.ds(j * 16, 16)]
          idx_s[pl.ds(j * 16, 16)] = v & 0xFFFF
          idx_d[pl.ds(j * 16, 16)] = lax.shift_right_logical(v, 16)
          return 0

        lax.fori_loop(0, K // 16, unpack, 0)
        pltpu.async_copy(feat_h.at[idx_s], rows, sem).wait()
        pltpu.sync_copy(rows, agg_sh.at[idx_d], add=True)
        if with_deg:
          pltpu.sync_copy(ones_v, deg_sh.at[idx_d], add=True)
        return 0

      lax.fori_loop(0, NCHUNK, chunk, 0)
      plsc.subcore_barrier()

      @pl.when(s < 10)
      def _():
        pltpu.sync_copy(agg_sh.at[pl.ds(s * 1000, 1000)],
                        agg_h.at[t, pl.ds(s * 1000, 1000)])
        if with_deg:
          pltpu.sync_copy(deg_sh.at[pl.ds(s * 1000, 1000)],
                          deg_h.at[t, pl.ds(s * 1000, 1000)])
      plsc.subcore_barrier()

  return pl.kernel(
      body, out_type=out_type, mesh=mesh, scratch_types=scratch,
      compiler_params=pltpu.CompilerParams(use_tc_tiling_on_sc=False))


def _sc_aggregate(feat_flat, pk_pad, with_deg):
  zrows = jnp.zeros((125, H), jnp.float32)
  zdeg = jnp.zeros((125, 8), jnp.float32)
  ones = jnp.ones((K, 8), jnp.float32)
  return _make_sc_agg(with_deg)(feat_flat, pk_pad, zrows, zdeg, ones)


def _p1_body(xs_ref, w_ref, b_ref, wl_ref, xm_ref, xp_ref):
  w = w_ref[...]
  b = b_ref[...]
  wl = wl_ref[...]
  xs = [xs_ref[t] for t in range(T)]
  for t in range(T):
    xm = xs[t] * w[1] + b
    if t > 0:
      xm = xm + xs[t - 1] * w[0]
    if t < T - 1:
      xm = xm + xs[t + 1] * w[2]
    xm_ref[t] = xm
    xp_ref[t] = jnp.dot(xm, wl, **_MM)


def _p1(xs, tm_w, tm_b, wl0):
  bV, grid = 2000, V // 2000
  w3 = jnp.transpose(tm_w[:, 0, :])
  return pl.pallas_call(
      _p1_body,
      grid=(grid,),
      in_specs=[
          pl.BlockSpec((T, bV, C), lambda i: (0, i, 0)),
          pl.BlockSpec((3, C), lambda i: (0, 0)),
          pl.BlockSpec((1, C), lambda i: (0, 0)),
          pl.BlockSpec((C, H), lambda i: (0, 0)),
      ],
      out_specs=[
          pl.BlockSpec((T, bV, C), lambda i: (0, i, 0)),
          pl.BlockSpec((T, bV, H), lambda i: (0, i, 0)),
      ],
      out_shape=[
          jax.ShapeDtypeStruct((T, V, C), jnp.float32),
          jax.ShapeDtypeStruct((T, V, H), jnp.float32),
      ],
  )(xs, w3, tm_b[None, :], wl0)


def _p3_body(xm_ref, agg_ref, deg_ref, wr_ref, rw_ref, sb_ref, rb_ref,
             av_ref, bv_ref, cw_ref, cb_ref, wl1_ref, out_ref, xp_ref):
  wr, rw, wl1 = wr_ref[...], rw_ref[...], wl1_ref[...]
  sb, rb, cb = sb_ref[...], rb_ref[...], cb_ref[...]
  av, bv = av_ref[...], bv_ref[...]
  cw = cw_ref[...]
  bV = xm_ref.shape[1]
  st = jnp.zeros((bV, HDS), jnp.float32)
  for t in range(T):
    xm = xm_ref[t]
    scale = 1.0 / jnp.maximum(deg_ref[t][:, 0:1], 1.0)
    h = agg_ref[t] * scale + jnp.dot(xm, wr, **_MM) + sb
    xsr = jnp.dot(xm, rw, **_MM) + rb
    h16 = jnp.concatenate([h] * DS, axis=1)
    st = av * st + bv * h16
    y = jnp.dot(jnp.maximum(st, 0.0), cw, **_MM) + cb
    out = y + xsr
    out_ref[t] = out
    xp_ref[t] = jnp.dot(out, wl1, **_MM)


def _p3(xs_m, agg0, deg, wr0, rw0, sb0, rb0, a0v, b0v, cw0r, cb0, wl1):
  bV, grid = 1000, V // 1000
  return pl.pallas_call(
      _p3_body,
      grid=(grid,),
      in_specs=[
          pl.BlockSpec((T, bV, C), lambda i: (0, i, 0)),
          pl.BlockSpec((T, bV, H), lambda i: (0, i, 0)),
          pl.BlockSpec((T, bV, 8), lambda i: (0, i, 0)),
          pl.BlockSpec((C, H), lambda i: (0, 0)),
          pl.BlockSpec((C, H), lambda i: (0, 0)),
          pl.BlockSpec((1, H), lambda i: (0, 0)),
          pl.BlockSpec((1, H), lambda i: (0, 0)),
          pl.BlockSpec((1, HDS), lambda i: (0, 0)),
          pl.BlockSpec((1, HDS), lambda i: (0, 0)),
          pl.BlockSpec((HDS, H), lambda i: (0, 0)),
          pl.BlockSpec((1, H), lambda i: (0, 0)),
          pl.BlockSpec((H, H), lambda i: (0, 0)),
      ],
      out_specs=[
          pl.BlockSpec((T, bV, H), lambda i: (0, i, 0)),
          pl.BlockSpec((T, bV, H), lambda i: (0, i, 0)),
      ],
      out_shape=[
          jax.ShapeDtypeStruct((T, V, H), jnp.float32),
          jax.ShapeDtypeStruct((T, V, H), jnp.float32),
      ],
  )(xs_m, agg0, deg, wr0, rw0, sb0, rb0, a0v, b0v, cw0r, cb0, wl1)


def _p5_body(x_ref, agg_ref, deg_ref, wr_ref, rw_ref, sb_ref, rb_ref,
             av_ref, bv_ref, cw_ref, cb_ref, mw_ref, mb_ref, out_ref):
  wr, rw = wr_ref[...], rw_ref[...]
  sb, rb, cb, mb = sb_ref[...], rb_ref[...], cb_ref[...], mb_ref[...]
  av, bv = av_ref[...], bv_ref[...]
  cw, mw = cw_ref[...], mw_ref[...]
  bV = x_ref.shape[1]
  st = jnp.zeros((bV, HDS), jnp.float32)
  for t in range(T):
    x = x_ref[t]
    scale = 1.0 / jnp.maximum(deg_ref[t][:, 0:1], 1.0)
    h = agg_ref[t] * scale + jnp.dot(x, wr, **_MM) + sb
    h16 = jnp.concatenate([h] * DS, axis=1)
    st = av * st + bv * h16
  y = jnp.dot(jnp.maximum(st, 0.0), cw, **_MM) + cb
  fin = y + jnp.dot(x_ref[T - 1], rw, **_MM) + rb
  out_ref[...] = jnp.dot(fin, mw, **_MM) + mb


def _p5(out0, agg1, deg, wr1, rw1, sb1, rb1, a1v, b1v, cw1r, cb1,
        mlp_W, mlp_b):
  bV, grid = 1000, V // 1000
  return pl.pallas_call(
      _p5_body,
      grid=(grid,),
      in_specs=[
          pl.BlockSpec((T, bV, H), lambda i: (0, i, 0)),
          pl.BlockSpec((T, bV, H), lambda i: (0, i, 0)),
          pl.BlockSpec((T, bV, 8), lambda i: (0, i, 0)),
          pl.BlockSpec((H, H), lambda i: (0, 0)),
          pl.BlockSpec((H, H), lambda i: (0, 0)),
          pl.BlockSpec((1, H), lambda i: (0, 0)),
          pl.BlockSpec((1, H), lambda i: (0, 0)),
          pl.BlockSpec((1, HDS), lambda i: (0, 0)),
          pl.BlockSpec((1, HDS), lambda i: (0, 0)),
          pl.BlockSpec((HDS, H), lambda i: (0, 0)),
          pl.BlockSpec((1, H), lambda i: (0, 0)),
          pl.BlockSpec((H, OUT), lambda i: (0, 0)),
          pl.BlockSpec((1, OUT), lambda i: (0, 0)),
      ],
      out_specs=pl.BlockSpec((bV, OUT), lambda i: (i, 0)),
      out_shape=jax.ShapeDtypeStruct((V, OUT), jnp.float32),
  )(out0, agg1, deg, wr1, rw1, sb1, rb1, a1v, b1v, cw1r, cb1, mlp_W,
    mlp_b[None, :])


def kernel(xs, edge_index, tm_w, tm_b, res_W0, res_b0, sage_Wl0, sage_Wr0,
           sage_b0, ssm_A0, ssm_B0, cm_W0, cm_b0, res_W1, res_b1, sage_Wl1,
           sage_Wr1, sage_b1, ssm_A1, ssm_B1, cm_W1, cm_b1, mlp_W, mlp_b):
  toff = (jnp.arange(T, dtype=jnp.int32) * V)[:, None]
  pk = (edge_index[:, 0, :] + toff) | (edge_index[:, 1, :] << 16)
  dummy = jnp.full((T, EPAD - E), V << 16, jnp.int32)
  pk_pad = jnp.concatenate([pk, dummy], axis=1).reshape(T * EPAD)

  a0v = jnp.repeat(ssm_A0, H)[None, :]
  b0v = jnp.repeat(ssm_B0, H)[None, :]
  a1v = jnp.repeat(ssm_A1, H)[None, :]
  b1v = jnp.repeat(ssm_B1, H)[None, :]
  cw0r = cm_W0.reshape(H, DS, H).transpose(1, 0, 2).reshape(HDS, H)
  cw1r = cm_W1.reshape(H, DS, H).transpose(1, 0, 2).reshape(HDS, H)

  xs_m, xp0 = _p1(xs, tm_w, tm_b, sage_Wl0)

  agg0, deg = _sc_aggregate(xp0.reshape(T * V, H), pk_pad, True)

  out0, xp1 = _p3(xs_m, agg0, deg, sage_Wr0, res_W0, sage_b0[None, :],
                  res_b0[None, :], a0v, b0v, cw0r, cm_b0[None, :], sage_Wl1)

  (agg1,) = _sc_aggregate(xp1.reshape(T * V, H), pk_pad, False)

  return _p5(out0, agg1, deg, sage_Wr1, res_W1, sage_b1[None, :],
             res_b1[None, :], a1v, b1v, cw1r, cm_b1[None, :], mlp_W, mlp_b)


# speedup vs baseline: 1.7433x; 1.7433x over previous
import functools

import jax
import jax.numpy as jnp
from jax import lax
from jax.experimental import pallas as pl
from jax.experimental.pallas import tpu as pltpu
from jax.experimental.pallas import tpu_sc as plsc

T, V, C, H, DS, E, OUT = 4, 10000, 128, 64, 16, 160000, 128
HDS = H * DS

NS = 16
K = 400
EP = E // NS
NCHUNK = EP // K

_MM = dict(preferred_element_type=jnp.float32,
           precision=jax.lax.Precision.DEFAULT)


@functools.cache
def _make_sc_agg(with_deg):
  mesh = plsc.VectorSubcoreMesh(core_axis_name="c", subcore_axis_name="s")
  out_type = [jax.ShapeDtypeStruct((T, V, H), jnp.float32)]
  if with_deg:
    out_type.append(jax.ShapeDtypeStruct((T, V, 8), jnp.float32))
  scratch = [
      pltpu.VMEM((K,), jnp.int32),
      pltpu.VMEM((K,), jnp.int32),
      pltpu.VMEM((K,), jnp.int32),
      pltpu.VMEM((K, H), jnp.float32),
      pltpu.VMEM((K, 8), jnp.float32),
      pltpu.SemaphoreType.DMA,
      pltpu.VMEM_SHARED((V, H), jnp.float32),
      pltpu.VMEM_SHARED((V, 8), jnp.float32),
  ]

  def body(feat_h, pk_h, zrows_h, zdeg_h, ones_h, *rest):
    if with_deg:
      agg_h, deg_h = rest[0], rest[1]
      rest = rest[2:]
    else:
      agg_h = rest[0]
      rest = rest[1:]
    idx_pk, idx_s, idx_d, rows, ones_v, sem, agg_sh, deg_sh = rest
    c = lax.axis_index("c")
    s = lax.axis_index("s")

    if with_deg:
      pltpu.sync_copy(ones_h, ones_v)

    for tl in range(2):
      t = 2 * c + tl

      @pl.when(s < 10)
      def _():
        for k2 in range(8):
          pltpu.sync_copy(zrows_h,
                          agg_sh.at[pl.ds(s * 1000 + k2 * 125, 125)])
          if with_deg:
            pltpu.sync_copy(zdeg_h,
                            deg_sh.at[pl.ds(s * 1000 + k2 * 125, 125)])
      plsc.subcore_barrier()

      def chunk(i, _):
        base = t * E + s * EP + i * K
        pltpu.sync_copy(pk_h.at[pl.ds(base, K)], idx_pk)

        def unpack(j, _2):
          v = idx_pk[pl.ds(j * 16, 16)]
          idx_s[pl.ds(j * 16, 16)] = v & 0xFFFF
          idx_d[pl.ds(j * 16, 16)] = lax.shift_right_logical(v, 16)
          return 0

        lax.fori_loop(0, K // 16, unpack, 0)
        pltpu.async_copy(feat_h.at[idx_s], rows, sem).wait()
        pltpu.sync_copy(rows, agg_sh.at[idx_d], add=True)
        if with_deg:
          pltpu.sync_copy(ones_v, deg_sh.at[idx_d], add=True)
        return 0

      lax.fori_loop(0, NCHUNK, chunk, 0)
      plsc.subcore_barrier()

      @pl.when(s < 10)
      def _():
        pltpu.sync_copy(agg_sh.at[pl.ds(s * 1000, 1000)],
                        agg_h.at[t, pl.ds(s * 1000, 1000)])
        if with_deg:
          pltpu.sync_copy(deg_sh.at[pl.ds(s * 1000, 1000)],
                          deg_h.at[t, pl.ds(s * 1000, 1000)])
      plsc.subcore_barrier()

  return pl.kernel(
      body, out_type=out_type, mesh=mesh, scratch_types=scratch,
      compiler_params=pltpu.CompilerParams(use_tc_tiling_on_sc=False))


def _sc_aggregate(feat_flat, pk_pad, with_deg):
  zrows = jnp.zeros((125, H), jnp.float32)
  zdeg = jnp.zeros((125, 8), jnp.float32)
  ones = jnp.ones((K, 8), jnp.float32)
  return _make_sc_agg(with_deg)(feat_flat, pk_pad, zrows, zdeg, ones)


def _p1_body(xs_ref, w_ref, b_ref, wl_ref, xm_ref, xp_ref):
  w = w_ref[...]
  b = b_ref[...]
  wl = wl_ref[...]
  xs = [xs_ref[t] for t in range(T)]
  for t in range(T):
    xm = xs[t] * w[1] + b
    if t > 0:
      xm = xm + xs[t - 1] * w[0]
    if t < T - 1:
      xm = xm + xs[t + 1] * w[2]
    xm_ref[t] = xm
    xp_ref[t] = jnp.dot(xm, wl, **_MM)


def _p1(xs, tm_w, tm_b, wl0):
  bV, grid = 2000, V // 2000
  w3 = jnp.transpose(tm_w[:, 0, :])
  return pl.pallas_call(
      _p1_body,
      grid=(grid,),
      in_specs=[
          pl.BlockSpec((T, bV, C), lambda i: (0, i, 0)),
          pl.BlockSpec((3, C), lambda i: (0, 0)),
          pl.BlockSpec((1, C), lambda i: (0, 0)),
          pl.BlockSpec((C, H), lambda i: (0, 0)),
      ],
      out_specs=[
          pl.BlockSpec((T, bV, C), lambda i: (0, i, 0)),
          pl.BlockSpec((T, bV, H), lambda i: (0, i, 0)),
      ],
      out_shape=[
          jax.ShapeDtypeStruct((T, V, C), jnp.float32),
          jax.ShapeDtypeStruct((T, V, H), jnp.float32),
      ],
  )(xs, w3, tm_b[None, :], wl0)


def _p3_body(xm_ref, agg_ref, deg_ref, wr_ref, rw_ref, sb_ref, rb_ref,
             av_ref, bv_ref, cw_ref, cb_ref, wl1_ref, out_ref, xp_ref):
  wr, rw, wl1 = wr_ref[...], rw_ref[...], wl1_ref[...]
  sb, rb, cb = sb_ref[...], rb_ref[...], cb_ref[...]
  av, bv = av_ref[...], bv_ref[...]
  cw = cw_ref[...]
  bV = xm_ref.shape[1]
  st = jnp.zeros((bV, HDS), jnp.float32)
  for t in range(T):
    xm = xm_ref[t]
    scale = 1.0 / jnp.maximum(deg_ref[t][:, 0:1], 1.0)
    h = agg_ref[t] * scale + jnp.dot(xm, wr, **_MM) + sb
    xsr = jnp.dot(xm, rw, **_MM) + rb
    h16 = jnp.concatenate([h] * DS, axis=1)
    st = av * st + bv * h16
    y = jnp.dot(jnp.maximum(st, 0.0), cw, **_MM) + cb
    out = y + xsr
    out_ref[t] = out
    xp_ref[t] = jnp.dot(out, wl1, **_MM)


def _p3(xs_m, agg0, deg, wr0, rw0, sb0, rb0, a0v, b0v, cw0r, cb0, wl1):
  bV, grid = 1000, V // 1000
  return pl.pallas_call(
      _p3_body,
      grid=(grid,),
      in_specs=[
          pl.BlockSpec((T, bV, C), lambda i: (0, i, 0)),
          pl.BlockSpec((T, bV, H), lambda i: (0, i, 0)),
          pl.BlockSpec((T, bV, 8), lambda i: (0, i, 0)),
          pl.BlockSpec((C, H), lambda i: (0, 0)),
          pl.BlockSpec((C, H), lambda i: (0, 0)),
          pl.BlockSpec((1, H), lambda i: (0, 0)),
          pl.BlockSpec((1, H), lambda i: (0, 0)),
          pl.BlockSpec((1, HDS), lambda i: (0, 0)),
          pl.BlockSpec((1, HDS), lambda i: (0, 0)),
          pl.BlockSpec((HDS, H), lambda i: (0, 0)),
          pl.BlockSpec((1, H), lambda i: (0, 0)),
          pl.BlockSpec((H, H), lambda i: (0, 0)),
      ],
      out_specs=[
          pl.BlockSpec((T, bV, H), lambda i: (0, i, 0)),
          pl.BlockSpec((T, bV, H), lambda i: (0, i, 0)),
      ],
      out_shape=[
          jax.ShapeDtypeStruct((T, V, H), jnp.float32),
          jax.ShapeDtypeStruct((T, V, H), jnp.float32),
      ],
  )(xs_m, agg0, deg, wr0, rw0, sb0, rb0, a0v, b0v, cw0r, cb0, wl1)


def _p5_body(x_ref, agg_ref, deg_ref, wr_ref, rw_ref, sb_ref, rb_ref,
             av_ref, bv_ref, cw_ref, cb_ref, mw_ref, mb_ref, out_ref):
  wr, rw = wr_ref[...], rw_ref[...]
  sb, rb, cb, mb = sb_ref[...], rb_ref[...], cb_ref[...], mb_ref[...]
  av, bv = av_ref[...], bv_ref[...]
  cw, mw = cw_ref[...], mw_ref[...]
  bV = x_ref.shape[1]
  st = jnp.zeros((bV, HDS), jnp.float32)
  for t in range(T):
    x = x_ref[t]
    scale = 1.0 / jnp.maximum(deg_ref[t][:, 0:1], 1.0)
    h = agg_ref[t] * scale + jnp.dot(x, wr, **_MM) + sb
    h16 = jnp.concatenate([h] * DS, axis=1)
    st = av * st + bv * h16
  y = jnp.dot(jnp.maximum(st, 0.0), cw, **_MM) + cb
  fin = y + jnp.dot(x_ref[T - 1], rw, **_MM) + rb
  out_ref[...] = jnp.dot(fin, mw, **_MM) + mb


def _p5(out0, agg1, deg, wr1, rw1, sb1, rb1, a1v, b1v, cw1r, cb1,
        mlp_W, mlp_b):
  bV, grid = 1000, V // 1000
  return pl.pallas_call(
      _p5_body,
      grid=(grid,),
      in_specs=[
          pl.BlockSpec((T, bV, H), lambda i: (0, i, 0)),
          pl.BlockSpec((T, bV, H), lambda i: (0, i, 0)),
          pl.BlockSpec((T, bV, 8), lambda i: (0, i, 0)),
          pl.BlockSpec((H, H), lambda i: (0, 0)),
          pl.BlockSpec((H, H), lambda i: (0, 0)),
          pl.BlockSpec((1, H), lambda i: (0, 0)),
          pl.BlockSpec((1, H), lambda i: (0, 0)),
          pl.BlockSpec((1, HDS), lambda i: (0, 0)),
          pl.BlockSpec((1, HDS), lambda i: (0, 0)),
          pl.BlockSpec((HDS, H), lambda i: (0, 0)),
          pl.BlockSpec((1, H), lambda i: (0, 0)),
          pl.BlockSpec((H, OUT), lambda i: (0, 0)),
          pl.BlockSpec((1, OUT), lambda i: (0, 0)),
      ],
      out_specs=pl.BlockSpec((bV, OUT), lambda i: (i, 0)),
      out_shape=jax.ShapeDtypeStruct((V, OUT), jnp.float32),
  )(out0, agg1, deg, wr1, rw1, sb1, rb1, a1v, b1v, cw1r, cb1, mlp_W,
    mlp_b[None, :])


def kernel(xs, edge_index, tm_w, tm_b, res_W0, res_b0, sage_Wl0, sage_Wr0,
           sage_b0, ssm_A0, ssm_B0, cm_W0, cm_b0, res_W1, res_b1, sage_Wl1,
           sage_Wr1, sage_b1, ssm_A1, ssm_B1, cm_W1, cm_b1, mlp_W, mlp_b):
  toff = (jnp.arange(T, dtype=jnp.int32) * V)[:, None]
  pk_pad = ((edge_index[:, 0, :] + toff)
            | (edge_index[:, 1, :] << 16)).reshape(T * E)

  a0v = jnp.repeat(ssm_A0, H)[None, :]
  b0v = jnp.repeat(ssm_B0, H)[None, :]
  a1v = jnp.repeat(ssm_A1, H)[None, :]
  b1v = jnp.repeat(ssm_B1, H)[None, :]
  cw0r = cm_W0.reshape(H, DS, H).transpose(1, 0, 2).reshape(HDS, H)
  cw1r = cm_W1.reshape(H, DS, H).transpose(1, 0, 2).reshape(HDS, H)

  xs_m, xp0 = _p1(xs, tm_w, tm_b, sage_Wl0)

  agg0, deg = _sc_aggregate(xp0.reshape(T * V, H), pk_pad, True)

  out0, xp1 = _p3(xs_m, agg0, deg, sage_Wr0, res_W0, sage_b0[None, :],
                  res_b0[None, :], a0v, b0v, cw0r, cm_b0[None, :], sage_Wl1)

  (agg1,) = _sc_aggregate(xp1.reshape(T * V, H), pk_pad, False)

  return _p5(out0, agg1, deg, sage_Wr1, res_W1, sage_b1[None, :],
             res_b1[None, :], a1v, b1v, cw1r, cm_b1[None, :], mlp_W, mlp_b)


# speedup vs baseline: 2.2458x; 1.2882x over previous
import functools

import jax
import jax.numpy as jnp
from jax import lax
from jax.experimental import pallas as pl
from jax.experimental.pallas import tpu as pltpu
from jax.experimental.pallas import tpu_sc as plsc

T, V, C, H, DS, E, OUT = 4, 10000, 128, 64, 16, 160000, 128
HDS = H * DS

NS = 16
K = 400
EP = E // NS
NCHUNK = EP // K

_MM = dict(preferred_element_type=jnp.float32,
           precision=jax.lax.Precision.DEFAULT)


@functools.cache
def _make_sc_agg(with_deg):
  mesh = plsc.VectorSubcoreMesh(core_axis_name="c", subcore_axis_name="s")
  out_type = [jax.ShapeDtypeStruct((T, V, H), jnp.float32)]
  if with_deg:
    out_type.append(jax.ShapeDtypeStruct((T, V, 8), jnp.float32))
  scratch = [
      pltpu.VMEM((EP,), jnp.int32),
      pltpu.VMEM((K,), jnp.int32),
      pltpu.VMEM((K,), jnp.int32),
      pltpu.VMEM((K, H), jnp.float32),
      pltpu.VMEM((K,), jnp.int32),
      pltpu.VMEM((K,), jnp.int32),
      pltpu.VMEM((K, H), jnp.float32),
      pltpu.VMEM((K, 8), jnp.float32),
      pltpu.SemaphoreType.DMA,
      pltpu.SemaphoreType.DMA,
      pltpu.VMEM_SHARED((V, H), jnp.float32),
      pltpu.VMEM_SHARED((V, 8), jnp.float32),
  ]

  def body(feat_h, pk_h, zrows_h, zdeg_h, ones_h, *rest):
    if with_deg:
      agg_h, deg_h = rest[0], rest[1]
      rest = rest[2:]
    else:
      agg_h = rest[0]
      rest = rest[1:]
    (idx_pk, idx_sa, idx_da, rows_a, idx_sb, idx_db, rows_b, ones_v,
     sem_a, sem_b, agg_sh, deg_sh) = rest
    c = lax.axis_index("c")
    s = lax.axis_index("s")

    if with_deg:
      pltpu.sync_copy(ones_h, ones_v)

    def start(ci, idx_s, idx_d, rows, sem):
      def unpack(j, _2):
        v = idx_pk[pl.ds(ci * K + j * 16, 16)]
        idx_s[pl.ds(j * 16, 16)] = v & 0xFFFF
        idx_d[pl.ds(j * 16, 16)] = lax.shift_right_logical(v, 16)
        return 0

      lax.fori_loop(0, K // 16, unpack, 0)
      pltpu.async_copy(feat_h.at[idx_s], rows, sem)

    def drain(idx_s, idx_d, rows, sem):
      pltpu.make_async_copy(feat_h.at[idx_s], rows, sem).wait()
      pltpu.sync_copy(rows, agg_sh.at[idx_d], add=True)
      if with_deg:
        pltpu.sync_copy(ones_v, deg_sh.at[idx_d], add=True)

    for tl in range(2):
      t = 2 * c + tl

      @pl.when(s < 10)
      def _():
        for k2 in range(8):
          pltpu.sync_copy(zrows_h,
                          agg_sh.at[pl.ds(s * 1000 + k2 * 125, 125)])
          if with_deg:
            pltpu.sync_copy(zdeg_h,
                            deg_sh.at[pl.ds(s * 1000 + k2 * 125, 125)])
      pltpu.sync_copy(pk_h.at[pl.ds(t * E + s * EP, EP)], idx_pk)
      plsc.subcore_barrier()

      start(0, idx_sa, idx_da, rows_a, sem_a)

      def pair(i, _):
        start(2 * i + 1, idx_sb, idx_db, rows_b, sem_b)
        drain(idx_sa, idx_da, rows_a, sem_a)
        start(2 * i + 2, idx_sa, idx_da, rows_a, sem_a)
        drain(idx_sb, idx_db, rows_b, sem_b)
        return 0

      lax.fori_loop(0, (NCHUNK - 1) // 2, pair, 0)
      drain(idx_sa, idx_da, rows_a, sem_a)
      plsc.subcore_barrier()

      @pl.when(s < 10)
      def _():
        pltpu.sync_copy(agg_sh.at[pl.ds(s * 1000, 1000)],
                        agg_h.at[t, pl.ds(s * 1000, 1000)])
        if with_deg:
          pltpu.sync_copy(deg_sh.at[pl.ds(s * 1000, 1000)],
                          deg_h.at[t, pl.ds(s * 1000, 1000)])
      plsc.subcore_barrier()

  return pl.kernel(
      body, out_type=out_type, mesh=mesh, scratch_types=scratch,
      compiler_params=pltpu.CompilerParams(use_tc_tiling_on_sc=False))


def _sc_aggregate(feat_flat, pk_pad, with_deg):
  zrows = jnp.zeros((125, H), jnp.float32)
  zdeg = jnp.zeros((125, 8), jnp.float32)
  ones = jnp.ones((K, 8), jnp.float32)
  return _make_sc_agg(with_deg)(feat_flat, pk_pad, zrows, zdeg, ones)


def _p1_body(xs_ref, w_ref, b_ref, wl_ref, xm_ref, xp_ref):
  w = w_ref[...]
  b = b_ref[...]
  wl = wl_ref[...]
  xs = [xs_ref[t] for t in range(T)]
  for t in range(T):
    xm = xs[t] * w[1] + b
    if t > 0:
      xm = xm + xs[t - 1] * w[0]
    if t < T - 1:
      xm = xm + xs[t + 1] * w[2]
    xm_ref[t] = xm
    xp_ref[t] = jnp.dot(xm, wl, **_MM)


def _p1(xs, tm_w, tm_b, wl0):
  bV, grid = 2000, V // 2000
  w3 = jnp.transpose(tm_w[:, 0, :])
  return pl.pallas_call(
      _p1_body,
      grid=(grid,),
      in_specs=[
          pl.BlockSpec((T, bV, C), lambda i: (0, i, 0)),
          pl.BlockSpec((3, C), lambda i: (0, 0)),
          pl.BlockSpec((1, C), lambda i: (0, 0)),
          pl.BlockSpec((C, H), lambda i: (0, 0)),
      ],
      out_specs=[
          pl.BlockSpec((T, bV, C), lambda i: (0, i, 0)),
          pl.BlockSpec((T, bV, H), lambda i: (0, i, 0)),
      ],
      out_shape=[
          jax.ShapeDtypeStruct((T, V, C), jnp.float32),
          jax.ShapeDtypeStruct((T, V, H), jnp.float32),
      ],
  )(xs, w3, tm_b[None, :], wl0)


def _p3_body(xm_ref, agg_ref, deg_ref, wr_ref, rw_ref, sb_ref, rb_ref,
             av_ref, bv_ref, cw_ref, cb_ref, wl1_ref, out_ref, xp_ref):
  wr, rw, wl1 = wr_ref[...], rw_ref[...], wl1_ref[...]
  sb, rb, cb = sb_ref[...], rb_ref[...], cb_ref[...]
  av, bv = av_ref[...], bv_ref[...]
  cw = cw_ref[...]
  bV = xm_ref.shape[1]
  st = jnp.zeros((bV, HDS), jnp.float32)
  for t in range(T):
    xm = xm_ref[t]
    scale = 1.0 / jnp.maximum(deg_ref[t][:, 0:1], 1.0)
    h = agg_ref[t] * scale + jnp.dot(xm, wr, **_MM) + sb
    xsr = jnp.dot(xm, rw, **_MM) + rb
    h16 = jnp.concatenate([h] * DS, axis=1)
    st = av * st + bv * h16
    y = jnp.dot(jnp.maximum(st, 0.0), cw, **_MM) + cb
    out = y + xsr
    out_ref[t] = out
    xp_ref[t] = jnp.dot(out, wl1, **_MM)


def _p3(xs_m, agg0, deg, wr0, rw0, sb0, rb0, a0v, b0v, cw0r, cb0, wl1):
  bV, grid = 1000, V // 1000
  return pl.pallas_call(
      _p3_body,
      grid=(grid,),
      in_specs=[
          pl.BlockSpec((T, bV, C), lambda i: (0, i, 0)),
          pl.BlockSpec((T, bV, H), lambda i: (0, i, 0)),
          pl.BlockSpec((T, bV, 8), lambda i: (0, i, 0)),
          pl.BlockSpec((C, H), lambda i: (0, 0)),
          pl.BlockSpec((C, H), lambda i: (0, 0)),
          pl.BlockSpec((1, H), lambda i: (0, 0)),
          pl.BlockSpec((1, H), lambda i: (0, 0)),
          pl.BlockSpec((1, HDS), lambda i: (0, 0)),
          pl.BlockSpec((1, HDS), lambda i: (0, 0)),
          pl.BlockSpec((HDS, H), lambda i: (0, 0)),
          pl.BlockSpec((1, H), lambda i: (0, 0)),
          pl.BlockSpec((H, H), lambda i: (0, 0)),
      ],
      out_specs=[
          pl.BlockSpec((T, bV, H), lambda i: (0, i, 0)),
          pl.BlockSpec((T, bV, H), lambda i: (0, i, 0)),
      ],
      out_shape=[
          jax.ShapeDtypeStruct((T, V, H), jnp.float32),
          jax.ShapeDtypeStruct((T, V, H), jnp.float32),
      ],
  )(xs_m, agg0, deg, wr0, rw0, sb0, rb0, a0v, b0v, cw0r, cb0, wl1)


def _p5_body(x_ref, agg_ref, deg_ref, wr_ref, rw_ref, sb_ref, rb_ref,
             av_ref, bv_ref, cw_ref, cb_ref, mw_ref, mb_ref, out_ref):
  wr, rw = wr_ref[...], rw_ref[...]
  sb, rb, cb, mb = sb_ref[...], rb_ref[...], cb_ref[...], mb_ref[...]
  av, bv = av_ref[...], bv_ref[...]
  cw, mw = cw_ref[...], mw_ref[...]
  bV = x_ref.shape[1]
  st = jnp.zeros((bV, HDS), jnp.float32)
  for t in range(T):
    x = x_ref[t]
    scale = 1.0 / jnp.maximum(deg_ref[t][:, 0:1], 1.0)
    h = agg_ref[t] * scale + jnp.dot(x, wr, **_MM) + sb
    h16 = jnp.concatenate([h] * DS, axis=1)
    st = av * st + bv * h16
  y = jnp.dot(jnp.maximum(st, 0.0), cw, **_MM) + cb
  fin = y + jnp.dot(x_ref[T - 1], rw, **_MM) + rb
  out_ref[...] = jnp.dot(fin, mw, **_MM) + mb


def _p5(out0, agg1, deg, wr1, rw1, sb1, rb1, a1v, b1v, cw1r, cb1,
        mlp_W, mlp_b):
  bV, grid = 1000, V // 1000
  return pl.pallas_call(
      _p5_body,
      grid=(grid,),
      in_specs=[
          pl.BlockSpec((T, bV, H), lambda i: (0, i, 0)),
          pl.BlockSpec((T, bV, H), lambda i: (0, i, 0)),
          pl.BlockSpec((T, bV, 8), lambda i: (0, i, 0)),
          pl.BlockSpec((H, H), lambda i: (0, 0)),
          pl.BlockSpec((H, H), lambda i: (0, 0)),
          pl.BlockSpec((1, H), lambda i: (0, 0)),
          pl.BlockSpec((1, H), lambda i: (0, 0)),
          pl.BlockSpec((1, HDS), lambda i: (0, 0)),
          pl.BlockSpec((1, HDS), lambda i: (0, 0)),
          pl.BlockSpec((HDS, H), lambda i: (0, 0)),
          pl.BlockSpec((1, H), lambda i: (0, 0)),
          pl.BlockSpec((H, OUT), lambda i: (0, 0)),
          pl.BlockSpec((1, OUT), lambda i: (0, 0)),
      ],
      out_specs=pl.BlockSpec((bV, OUT), lambda i: (i, 0)),
      out_shape=jax.ShapeDtypeStruct((V, OUT), jnp.float32),
  )(out0, agg1, deg, wr1, rw1, sb1, rb1, a1v, b1v, cw1r, cb1, mlp_W,
    mlp_b[None, :])


def kernel(xs, edge_index, tm_w, tm_b, res_W0, res_b0, sage_Wl0, sage_Wr0,
           sage_b0, ssm_A0, ssm_B0, cm_W0, cm_b0, res_W1, res_b1, sage_Wl1,
           sage_Wr1, sage_b1, ssm_A1, ssm_B1, cm_W1, cm_b1, mlp_W, mlp_b):
  toff = (jnp.arange(T, dtype=jnp.int32) * V)[:, None]
  pk_pad = ((edge_index[:, 0, :] + toff)
            | (edge_index[:, 1, :] << 16)).reshape(T * E)

  a0v = jnp.repeat(ssm_A0, H)[None, :]
  b0v = jnp.repeat(ssm_B0, H)[None, :]
  a1v = jnp.repeat(ssm_A1, H)[None, :]
  b1v = jnp.repeat(ssm_B1, H)[None, :]
  cw0r = cm_W0.reshape(H, DS, H).transpose(1, 0, 2).reshape(HDS, H)
  cw1r = cm_W1.reshape(H, DS, H).transpose(1, 0, 2).reshape(HDS, H)

  xs_m, xp0 = _p1(xs, tm_w, tm_b, sage_Wl0)

  agg0, deg = _sc_aggregate(xp0.reshape(T * V, H), pk_pad, True)

  out0, xp1 = _p3(xs_m, agg0, deg, sage_Wr0, res_W0, sage_b0[None, :],
                  res_b0[None, :], a0v, b0v, cw0r, cm_b0[None, :], sage_Wl1)

  (agg1,) = _sc_aggregate(xp1.reshape(T * V, H), pk_pad, False)

  return _p5(out0, agg1, deg, sage_Wr1, res_W1, sage_b1[None, :],
             res_b1[None, :], a1v, b1v, cw1r, cm_b1[None, :], mlp_W, mlp_b)
